# baseline (speedup 1.0000x reference)
import functools

import jax
import jax.numpy as jnp
from jax import lax
from jax.experimental import pallas as pl
from jax.experimental.pallas import tpu as pltpu
from jax.experimental.pallas import tpu_sc as plsc

N_NODES = 100000
H_DIM = 128
NUM_RELS = 100
N_TRIPLETS = 500000

NC = 2
NS = 16
L = 16
NW = NC * NS

C = 128
TU = 2
W = H_DIM // 2
NSLOT = 3


def _score_body(n_chunks, b_per_w,
                emb_hbm, wrel_hbm, src_hbm, rel_hbm, dst_hbm, out_hbm,
                wrel_v, sidx_v, didx_v, ridx_v, srows_v, orows_v, scores_v,
                sem_idx, sem_rows, sem_out):
    wid = lax.axis_index("s") * NC + lax.axis_index("c")
    base_w = wid * b_per_w
    pltpu.sync_copy(wrel_hbm, wrel_v)
    lane = lax.iota(jnp.int32, L)

    def start_idx(chunk, slot):
        base = base_w + chunk * C
        pltpu.async_copy(src_hbm.at[pl.ds(base, C)], sidx_v[slot], sem_idx[slot])
        pltpu.async_copy(dst_hbm.at[pl.ds(base, C)], didx_v[slot], sem_idx[slot])
        pltpu.async_copy(rel_hbm.at[pl.ds(base, C)], ridx_v[slot].at[pl.ds(0, C)],
                         sem_idx[slot])

    def wait_idx(slot):
        pltpu.make_async_copy(src_hbm.at[pl.ds(0, C)], sidx_v[slot], sem_idx[slot]).wait()
        pltpu.make_async_copy(dst_hbm.at[pl.ds(0, C)], didx_v[slot], sem_idx[slot]).wait()
        pltpu.make_async_copy(rel_hbm.at[pl.ds(0, C)], ridx_v[slot].at[pl.ds(0, C)],
                              sem_idx[slot]).wait()

    def start_rows(slot):
        pltpu.async_copy(emb_hbm.at[sidx_v[slot]], srows_v[slot], sem_rows[slot])
        pltpu.async_copy(emb_hbm.at[didx_v[slot]], orows_v[slot], sem_rows[slot])

    def wait_rows(slot):
        pltpu.make_async_copy(emb_hbm.at[sidx_v[slot]], srows_v[slot], sem_rows[slot]).wait()
        pltpu.make_async_copy(emb_hbm.at[didx_v[slot]], orows_v[slot], sem_rows[slot]).wait()

    def start_out(chunk, slot):
        pltpu.async_copy(scores_v[slot], out_hbm.at[pl.ds(base_w + chunk * C, C)],
                         sem_out[slot])

    def wait_out(slot):
        pltpu.make_async_copy(scores_v[slot], out_hbm.at[pl.ds(0, C)], sem_out[slot]).wait()

    def compute(slot):
        zero = jnp.zeros((L,), jnp.float32)

        def zero_body(g, gcarry):
            scores_v[slot][pl.ds(g * L, L)] = zero
            return gcarry

        lax.fori_loop(0, C // L, zero_body, 0)

        def trip_body(tt, tcarry):
            rvec = ridx_v[slot][pl.ds(tt * TU, L)] * W
            for u in range(TU):
                t = tt * TU + u
                rbase = rvec[u]
                acc0 = zero
                acc1 = zero
                for k in range(W // L):
                    sv = srows_v[slot][t, pl.ds(k * L, L)]
                    ov = orows_v[slot][t, pl.ds(k * L, L)]
                    rv = wrel_v[pl.ds(rbase + k * L, L)]
                    prod = (plsc.bitcast(sv, jnp.bfloat16)
                            * plsc.bitcast(ov, jnp.bfloat16)
                            * plsc.bitcast(rv, jnp.bfloat16))
                    pi = plsc.bitcast(prod, jnp.int32)
                    pe = plsc.bitcast(pi << 16, jnp.float32)
                    po = plsc.bitcast(pi & jnp.int32(-65536), jnp.float32)
                    acc0 = acc0 + pe
                    acc1 = acc1 + po
                plsc.addupdate_scatter(scores_v[slot],
                                       [jnp.full((L,), t, jnp.int32)],
                                       acc0 + acc1)
            return tcarry

        lax.fori_loop(0, C // TU, trip_body, 0)

    for s in range(NSLOT):
        start_idx(s, s)
    wait_idx(0)
    start_rows(0)
    for i in range(NSLOT):
        wait_idx((i + 1) % NSLOT)
        start_rows((i + 1) % NSLOT)
        wait_rows(i)
        compute(i)
        start_out(i, i)
        start_idx(i + NSLOT, i)

    def block_body(g, carry):
        for k in range(NSLOT):
            i = NSLOT + NSLOT * g + k
            wait_idx((k + 1) % NSLOT)
            start_rows((k + 1) % NSLOT)
            wait_rows(k)
            wait_out(k)
            compute(k)
            start_out(i, k)
            start_idx(i + NSLOT, k)
        return carry

    lax.fori_loop(0, (n_chunks - 2 * NSLOT) // NSLOT, block_body, 0)

    i = n_chunks - NSLOT
    for k in range(NSLOT - 1):
        wait_idx(k + 1)
        start_rows(k + 1)
        wait_rows(k)
        wait_out(k)
        compute(k)
        start_out(i + k, k)
    wait_rows(NSLOT - 1)
    wait_out(NSLOT - 1)
    compute(NSLOT - 1)
    start_out(n_chunks - 1, NSLOT - 1)
    for s in range(NSLOT):
        wait_out(s)


def kernel(embedding, w_relation, src, rel, dst):
    n = src.shape[0]
    step = NW * C * NSLOT
    n_pad = ((n + step - 1) // step) * step
    pad = n_pad - n
    if pad:
        zpad = jnp.zeros((pad,), src.dtype)
        src = jnp.concatenate([src, zpad])
        rel = jnp.concatenate([rel, zpad])
        dst = jnp.concatenate([dst, zpad])
    b_per_w = n_pad // NW
    n_chunks = b_per_w // C
    assert n_chunks >= 3 * NSLOT and n_chunks % NSLOT == 0

    def _pack_bf16_pairs(x):
        u = jax.lax.bitcast_convert_type(x, jnp.uint32)
        hi = (u + jnp.uint32(0x7FFF) + ((u >> 16) & jnp.uint32(1))) >> 16
        packed = hi[:, 0::2] | (hi[:, 1::2] << 16)
        return jax.lax.bitcast_convert_type(packed, jnp.float32)

    emb_packed = _pack_bf16_pairs(embedding)
    wrel_packed = _pack_bf16_pairs(w_relation)

    mesh = plsc.VectorSubcoreMesh(core_axis_name="c", subcore_axis_name="s")
    body = functools.partial(_score_body, n_chunks, b_per_w)
    score = pl.kernel(
        body,
        out_type=jax.ShapeDtypeStruct((n_pad,), jnp.float32),
        mesh=mesh,
        compiler_params=pltpu.CompilerParams(needs_layout_passes=False, use_tc_tiling_on_sc=False),
        scratch_types=[
            pltpu.VMEM((NUM_RELS * W,), jnp.float32),
            [pltpu.VMEM((C,), jnp.int32) for _ in range(NSLOT)],
            [pltpu.VMEM((C,), jnp.int32) for _ in range(NSLOT)],
            [pltpu.VMEM((C + L,), jnp.int32) for _ in range(NSLOT)],
            [pltpu.VMEM((C, W), jnp.float32) for _ in range(NSLOT)],
            [pltpu.VMEM((C, W), jnp.float32) for _ in range(NSLOT)],
            [pltpu.VMEM((C,), jnp.float32) for _ in range(NSLOT)],
            [pltpu.SemaphoreType.DMA for _ in range(NSLOT)],
            [pltpu.SemaphoreType.DMA for _ in range(NSLOT)],
            [pltpu.SemaphoreType.DMA for _ in range(NSLOT)],
        ],
    )(emb_packed, wrel_packed.reshape(-1), src, rel, dst)
    return score[:n]


# speedup vs baseline: 3.3527x; 3.3527x over previous
import functools

import jax
import jax.numpy as jnp
from jax import lax
from jax.experimental import pallas as pl
from jax.experimental.pallas import tpu as pltpu
from jax.experimental.pallas import tpu_sc as plsc

N_NODES = 100000
H_DIM = 128
NUM_RELS = 100
N_TRIPLETS = 500000

NC = 2
NS = 16
L = 16
NW = NC * NS

C = 128
TU = 2
W = H_DIM // 2
NSLOT = 3


def _score_body(n_chunks, b_per_w,
                emb_hbm, wrel_hbm, src_hbm, rel_hbm, dst_hbm, out_hbm,
                wrel_v, sidx_v, didx_v, ridx_v, srows_v, orows_v, scores_v,
                sem_idx, sem_rows, sem_out):
    wid = lax.axis_index("s") * NC + lax.axis_index("c")
    base_w = wid * b_per_w
    pltpu.sync_copy(wrel_hbm, wrel_v)
    lane = lax.iota(jnp.int32, L)

    def start_idx(chunk, slot):
        base = base_w + chunk * C
        pltpu.async_copy(src_hbm.at[pl.ds(base, C)], sidx_v[slot], sem_idx[slot])
        pltpu.async_copy(dst_hbm.at[pl.ds(base, C)], didx_v[slot], sem_idx[slot])
        pltpu.async_copy(rel_hbm.at[pl.ds(base, C)], ridx_v[slot].at[pl.ds(0, C)],
                         sem_idx[slot])

    def wait_idx(slot):
        pltpu.make_async_copy(src_hbm.at[pl.ds(0, C)], sidx_v[slot], sem_idx[slot]).wait()
        pltpu.make_async_copy(dst_hbm.at[pl.ds(0, C)], didx_v[slot], sem_idx[slot]).wait()
        pltpu.make_async_copy(rel_hbm.at[pl.ds(0, C)], ridx_v[slot].at[pl.ds(0, C)],
                              sem_idx[slot]).wait()

    def start_rows(slot):
        pltpu.async_copy(emb_hbm.at[sidx_v[slot]], srows_v[slot], sem_rows[slot])
        pltpu.async_copy(emb_hbm.at[didx_v[slot]], orows_v[slot], sem_rows[slot])

    def wait_rows(slot):
        pltpu.make_async_copy(emb_hbm.at[sidx_v[slot]], srows_v[slot], sem_rows[slot]).wait()
        pltpu.make_async_copy(emb_hbm.at[didx_v[slot]], orows_v[slot], sem_rows[slot]).wait()

    def start_out(chunk, slot):
        pltpu.async_copy(scores_v[slot], out_hbm.at[pl.ds(base_w + chunk * C, C)],
                         sem_out[slot])

    def wait_out(slot):
        pltpu.make_async_copy(scores_v[slot], out_hbm.at[pl.ds(0, C)], sem_out[slot]).wait()

    def compute(slot):
        zero = jnp.zeros((L,), jnp.float32)

        def zero_body(g, gcarry):
            scores_v[slot][pl.ds(g * L, L)] = zero
            return gcarry

        lax.fori_loop(0, C // L, zero_body, 0)

        def trip_body(tt, tcarry):
            rvec = ridx_v[slot][pl.ds(tt * TU, L)] * W
            for u in range(TU):
                t = tt * TU + u
                rbase = rvec[u]
                acc0 = zero
                acc1 = zero
                for k in range(W // L):
                    sv = srows_v[slot][t, pl.ds(k * L, L)]
                    ov = orows_v[slot][t, pl.ds(k * L, L)]
                    rv = wrel_v[pl.ds(rbase + k * L, L)]
                    prod = (plsc.bitcast(sv, jnp.bfloat16)
                            * plsc.bitcast(ov, jnp.bfloat16)
                            * plsc.bitcast(rv, jnp.bfloat16))
                    pi = plsc.bitcast(prod, jnp.int32)
                    pe = plsc.bitcast(pi << 16, jnp.float32)
                    po = plsc.bitcast(pi & jnp.int32(-65536), jnp.float32)
                    acc0 = acc0 + pe
                    acc1 = acc1 + po
                plsc.addupdate_scatter(scores_v[slot],
                                       [jnp.full((L,), t, jnp.int32)],
                                       acc0 + acc1)
            return tcarry

        lax.fori_loop(0, C // TU, trip_body, 0)

    for s in range(NSLOT):
        start_idx(s, s)
    wait_idx(0)
    start_rows(0)
    for i in range(NSLOT):
        wait_idx((i + 1) % NSLOT)
        start_rows((i + 1) % NSLOT)
        wait_rows(i)
        compute(i)
        start_out(i, i)
        start_idx(i + NSLOT, i)

    def block_body(g, carry):
        for k in range(NSLOT):
            i = NSLOT + NSLOT * g + k
            wait_idx((k + 1) % NSLOT)
            start_rows((k + 1) % NSLOT)
            wait_rows(k)
            wait_out(k)
            compute(k)
            start_out(i, k)
            start_idx(i + NSLOT, k)
        return carry

    lax.fori_loop(0, (n_chunks - 2 * NSLOT) // NSLOT, block_body, 0)

    i = n_chunks - NSLOT
    for k in range(NSLOT - 1):
        wait_idx(k + 1)
        start_rows(k + 1)
        wait_rows(k)
        wait_out(k)
        compute(k)
        start_out(i + k, k)
    wait_rows(NSLOT - 1)
    wait_out(NSLOT - 1)
    compute(NSLOT - 1)
    start_out(n_chunks - 1, NSLOT - 1)
    for s in range(NSLOT):
        wait_out(s)


def kernel(embedding, w_relation, src, rel, dst):
    n = src.shape[0]
    step = NW * C * NSLOT
    n_pad = ((n + step - 1) // step) * step
    pad = n_pad - n
    if pad:
        zpad = jnp.zeros((pad,), src.dtype)
        src = jnp.concatenate([src, zpad])
        rel = jnp.concatenate([rel, zpad])
        dst = jnp.concatenate([dst, zpad])
    b_per_w = n_pad // NW
    n_chunks = b_per_w // C
    assert n_chunks >= 3 * NSLOT and n_chunks % NSLOT == 0

    def _pack_bf16_pairs(x):
        u = jax.lax.bitcast_convert_type(x, jnp.uint32)
        hi = (u + jnp.uint32(0x7FFF) + ((u >> 16) & jnp.uint32(1))) >> 16
        packed = hi[:, :W] | (hi[:, W:] << 16)
        return jax.lax.bitcast_convert_type(packed, jnp.float32)

    emb_packed = _pack_bf16_pairs(embedding)
    wrel_packed = _pack_bf16_pairs(w_relation)

    mesh = plsc.VectorSubcoreMesh(core_axis_name="c", subcore_axis_name="s")
    body = functools.partial(_score_body, n_chunks, b_per_w)
    score = pl.kernel(
        body,
        out_type=jax.ShapeDtypeStruct((n_pad,), jnp.float32),
        mesh=mesh,
        compiler_params=pltpu.CompilerParams(needs_layout_passes=False, use_tc_tiling_on_sc=False),
        scratch_types=[
            pltpu.VMEM((NUM_RELS * W,), jnp.float32),
            [pltpu.VMEM((C,), jnp.int32) for _ in range(NSLOT)],
            [pltpu.VMEM((C,), jnp.int32) for _ in range(NSLOT)],
            [pltpu.VMEM((C + L,), jnp.int32) for _ in range(NSLOT)],
            [pltpu.VMEM((C, W), jnp.float32) for _ in range(NSLOT)],
            [pltpu.VMEM((C, W), jnp.float32) for _ in range(NSLOT)],
            [pltpu.VMEM((C,), jnp.float32) for _ in range(NSLOT)],
            [pltpu.SemaphoreType.DMA for _ in range(NSLOT)],
            [pltpu.SemaphoreType.DMA for _ in range(NSLOT)],
            [pltpu.SemaphoreType.DMA for _ in range(NSLOT)],
        ],
    )(emb_packed, wrel_packed.reshape(-1), src, rel, dst)
    return score[:n]
